# baseline (speedup 1.0000x reference)
import functools

import jax
import jax.numpy as jnp
from jax import lax
from jax.experimental import pallas as pl
from jax.experimental.pallas import tpu as pltpu
from jax.experimental.pallas import tpu_sc as plsc

_SPLIT = 128
_CH = 32
_NBUF = 8
_HG = 16


def _sc_gather_means(obj_id, obj_size, lines_idx, hist_idx,
                     obj_table, size_table, hist_table):
    B, L = lines_idx.shape
    Hl = hist_idx.shape[1]
    Vh = hist_table.shape[0]
    D = obj_table.shape[1]
    info = plsc.get_sparse_core_info()
    NC, NS = info.num_cores, info.num_subcores
    NW = NC * NS
    RPT = B // NW
    L2 = L - _SPLIT
    inv_l = 1.0 / L
    inv_h = 1.0 / Hl

    mesh = plsc.VectorSubcoreMesh(core_axis_name="c", subcore_axis_name="s")
    f32 = jnp.float32
    out_sds = jax.ShapeDtypeStruct((B, D), f32)

    @functools.partial(
        pl.kernel,
        mesh=mesh,
        out_type=(out_sds, out_sds, out_sds, out_sds),
        compiler_params=pltpu.CompilerParams(use_tc_tiling_on_sc=False,
                                             needs_layout_passes=False),
        scratch_types=[
            pltpu.VMEM((2, _CH, L), jnp.int32),
            pltpu.VMEM((2, _CH, Hl), jnp.int32),
            pltpu.VMEM((_CH,), jnp.int32),
            pltpu.VMEM((_CH,), jnp.int32),
            pltpu.VMEM((_NBUF, L, D), f32),
            pltpu.VMEM((Vh, D), f32),
            pltpu.VMEM((_CH, D), f32),
            pltpu.VMEM((_CH, D), f32),
            pltpu.VMEM((_CH, D), f32),
            pltpu.VMEM((_CH, D), f32),
        ] + [pltpu.SemaphoreType.DMA] * (_NBUF + 1),
    )
    def k(oid_h, osz_h, lix_h, hix_h, otab_h, stab_h, htab_h,
          id_out, sz_out, lm_out, hm_out,
          lix_v, hix_v, oid_v, osz_v, lbuf, htab_v, idbuf, szbuf, lmv, hmv,
          *allsems):
        sems = list(allsems[:_NBUF])
        sem_e = allsems[_NBUF]
        wid = lax.axis_index("s") * NC + lax.axis_index("c")
        base = wid * RPT

        pltpu.sync_copy(htab_h, htab_v)

        def stage(c):
            scd = c % 2
            row0 = base + c * _CH
            pltpu.sync_copy(lix_h.at[pl.ds(row0, _CH)], lix_v.at[scd])
            pltpu.sync_copy(hix_h.at[pl.ds(row0, _CH)], hix_v.at[scd])
            pltpu.sync_copy(oid_h.at[pl.ds(row0, _CH)], oid_v)
            pltpu.sync_copy(osz_h.at[pl.ds(row0, _CH)], osz_v)
            ca = pltpu.async_copy(otab_h.at[oid_v], idbuf, sem_e)
            cb = pltpu.async_copy(stab_h.at[osz_v], szbuf, sem_e)
            ca.wait()
            cb.wait()
            pltpu.sync_copy(idbuf, id_out.at[pl.ds(row0, _CH)])
            pltpu.sync_copy(szbuf, sz_out.at[pl.ds(row0, _CH)])

        def row_copies(r, b):
            scd = (r // _CH) % 2
            rr = r % _CH
            return tuple(
                pltpu.make_async_copy(
                    otab_h.at[lix_v.at[scd, rr, pl.ds(o, n)]],
                    lbuf.at[b, pl.ds(o, n)], sems[b])
                for o, n in ((0, _SPLIT), (_SPLIT, L2)))

        def issue(r, b):
            for cp in row_copies(r, b):
                cp.start()

        def wait(r, b):
            for cp in row_copies(r, b):
                cp.wait()

        stage(0)
        for b in range(_NBUF - 1):
            issue(b, b)

        z = jnp.zeros((16,), f32)
        lanes = lax.iota(jnp.int32, 16)

        def hist_group(r):
            scd = (r // _CH) % 2
            rr0 = r - (_HG - 1)
            rows = (rr0 % _CH) + lanes
            scds = jnp.full((16,), scd, jnp.int32)

            def hj(j, accs):
                js = jnp.full((16,), j, jnp.int32)
                hvec = plsc.load_gather(hix_v, [scds, rows, js])
                return tuple(
                    accs[c] + plsc.load_gather(
                        htab_v, [hvec, jnp.full((16,), c, jnp.int32)])
                    for c in range(D))

            accs = lax.fori_loop(0, Hl, hj, (z,) * D)
            out_rows = (rr0 % _CH) + lanes
            for c in range(D):
                plsc.store_scatter(hmv,
                                   [out_rows, jnp.full((16,), c, jnp.int32)],
                                   accs[c] * inv_h)

        def outer(i, _):
            for b in range(_NBUF):
                r = i * _NBUF + b
                pre = r + _NBUF - 1
                pb = (b + _NBUF - 1) % _NBUF
                wait(r, b)

                @pl.when(jnp.logical_and(pre % _CH == 0, pre < RPT))
                def _():
                    stage(pre // _CH)

                @pl.when(pre < RPT)
                def _():
                    issue(pre, pb)

                def acc_l(j8, carry):
                    accs = list(carry)
                    for u in range(8):
                        j = j8 * 8 + u
                        accs[u % 4] = accs[u % 4] + lbuf[b, j, pl.ds(0, 16)]
                        accs[4 + u % 4] = (accs[4 + u % 4]
                                           + lbuf[b, j, pl.ds(16, 16)])
                    return tuple(accs)

                accs = lax.fori_loop(0, L // 8, acc_l, (z,) * 8)
                a0 = (accs[0] + accs[1]) + (accs[2] + accs[3])
                a1 = (accs[4] + accs[5]) + (accs[6] + accs[7])
                lmv[r % _CH, pl.ds(0, 16)] = a0 * inv_l
                lmv[r % _CH, pl.ds(16, 16)] = a1 * inv_l

                @pl.when(r % _HG == _HG - 1)
                def _():
                    hist_group(r)

                @pl.when(r % _CH == _CH - 1)
                def _():
                    row0 = base + (r // _CH) * _CH
                    pltpu.sync_copy(lmv, lm_out.at[pl.ds(row0, _CH)])
                    pltpu.sync_copy(hmv, hm_out.at[pl.ds(row0, _CH)])
            return 0

        lax.fori_loop(0, RPT // _NBUF, outer, 0)

    return k(obj_id, obj_size, lines_idx, hist_idx,
             obj_table, size_table, hist_table)


def _sc_transpose(tabT):
    D, V = tabT.shape
    N = 1600
    NCHUNK = V // N
    info = plsc.get_sparse_core_info()
    NC, NS = info.num_cores, info.num_subcores
    NW = NC * NS
    PER = (NCHUNK + NW - 1) // NW
    mesh = plsc.VectorSubcoreMesh(core_axis_name="c", subcore_axis_name="s")

    @functools.partial(
        pl.kernel,
        mesh=mesh,
        out_type=jax.ShapeDtypeStruct((V, D), jnp.float32),
        compiler_params=pltpu.CompilerParams(use_tc_tiling_on_sc=False,
                                             needs_layout_passes=False),
        scratch_types=[
            pltpu.VMEM((D, N), jnp.float32),
            pltpu.VMEM((N, D), jnp.float32),
            pltpu.SemaphoreType.DMA,
        ],
    )
    def kt(tabT_h, out_h, pv, ov, sem):
        wid = lax.axis_index("s") * NC + lax.axis_index("c")
        lanes = lax.iota(jnp.int32, 16)

        def chunk_body(k, _):
            cid = k * NW + wid

            @pl.when(cid < NCHUNK)
            def _():
                off = cid * N
                cps = [pltpu.make_async_copy(
                    tabT_h.at[c, pl.ds(off, N)], pv.at[c], sem)
                    for c in range(D)]
                for cp in cps:
                    cp.start()
                for cp in cps:
                    cp.wait()

                def rows16(r0v, _):
                    r0 = r0v * 16
                    for g in range(D // 16):
                        cl = lanes + g * 16
                        for u in range(16):
                            val = plsc.load_gather(
                                pv, [cl, jnp.full((16,), r0 + u, jnp.int32)])
                            ov[r0 + u, pl.ds(g * 16, 16)] = val
                    return 0

                lax.fori_loop(0, N // 16, rows16, 0)
                pltpu.sync_copy(ov, out_h.at[pl.ds(off, N)])
            return 0

        lax.fori_loop(0, PER, chunk_body, 0)

    return kt(tabT)


def _lstm_tc_body(id_ref, sz_ref, lm_ref, hm_ref, w1_ref, w2_ref, b_ref,
                  out_ref):
    g = (jnp.dot(id_ref[...], w1_ref[...], preferred_element_type=jnp.float32)
         + jnp.dot(sz_ref[...], w2_ref[...], preferred_element_type=jnp.float32)
         + b_ref[...])
    Hh = g.shape[1] // 3
    i = jax.nn.sigmoid(g[:, :Hh])
    gg = jnp.tanh(g[:, Hh:2 * Hh])
    o = jax.nn.sigmoid(g[:, 2 * Hh:])
    h1 = o * jnp.tanh(i * gg)
    out_ref[:, :Hh] = h1
    D = lm_ref.shape[1]
    out_ref[:, Hh:Hh + D] = lm_ref[...]
    out_ref[:, Hh + D:Hh + 2 * D] = hm_ref[...]


def kernel(obj_id, obj_size, cache_lines, cache_history, obj_id_table,
           obj_size_table, history_table, W_ih, W_hh, b_ih, b_hh):
    B = obj_id.shape[0]
    D = obj_id_table.shape[1]
    Hh = W_hh.shape[1]

    tab_rm = _sc_transpose(obj_id_table.T)
    id_emb, sz_emb, lines_mean, hist_mean = _sc_gather_means(
        obj_id, obj_size, cache_lines, cache_history,
        tab_rm, obj_size_table, history_table)

    Wt = W_ih.T
    Wk = jnp.concatenate([Wt[:, :Hh], Wt[:, 2 * Hh:]], axis=1)
    bk = (b_ih + b_hh)
    bk = jnp.concatenate([bk[:Hh], bk[2 * Hh:]])[None, :]
    w1, w2 = Wk[:D], Wk[D:]

    BM = 2048
    grid = (B // BM,)
    out = pl.pallas_call(
        _lstm_tc_body,
        grid=grid,
        in_specs=[
            pl.BlockSpec((BM, D), lambda i: (i, 0)),
            pl.BlockSpec((BM, D), lambda i: (i, 0)),
            pl.BlockSpec((BM, D), lambda i: (i, 0)),
            pl.BlockSpec((BM, D), lambda i: (i, 0)),
            pl.BlockSpec((D, 3 * Hh), lambda i: (0, 0)),
            pl.BlockSpec((D, 3 * Hh), lambda i: (0, 0)),
            pl.BlockSpec((1, 3 * Hh), lambda i: (0, 0)),
        ],
        out_specs=pl.BlockSpec((BM, Hh + 2 * D), lambda i: (i, 0)),
        out_shape=jax.ShapeDtypeStruct((B, Hh + 2 * D), jnp.float32),
    )(id_emb, sz_emb, lines_mean, hist_mean, w1, w2, bk)
    return out


# speedup vs baseline: 3.5460x; 3.5460x over previous
import functools

import jax
import jax.numpy as jnp
from jax import lax
from jax.experimental import pallas as pl
from jax.experimental.pallas import tpu as pltpu
from jax.experimental.pallas import tpu_sc as plsc

_SPLIT = 128
_CH = 32
_NBUF = 8
_HG = 16


def _sc_gather_means(obj_id, obj_size, lines_idx, hist_idx,
                     obj_table, size_table, hist_table):
    B, L = lines_idx.shape
    Hl = hist_idx.shape[1]
    Vh = hist_table.shape[0]
    D = obj_table.shape[1]
    info = plsc.get_sparse_core_info()
    NC, NS = info.num_cores, info.num_subcores
    NW = NC * NS
    RPT = B // NW
    L2 = L - _SPLIT
    inv_l = 1.0 / L
    inv_h = 1.0 / Hl

    mesh = plsc.VectorSubcoreMesh(core_axis_name="c", subcore_axis_name="s")
    f32 = jnp.float32
    out_sds = jax.ShapeDtypeStruct((B, D), f32)

    @functools.partial(
        pl.kernel,
        mesh=mesh,
        out_type=(out_sds, out_sds, out_sds, out_sds),
        compiler_params=pltpu.CompilerParams(use_tc_tiling_on_sc=False,
                                             needs_layout_passes=False),
        scratch_types=[
            pltpu.VMEM((2, _CH, L), jnp.int32),
            pltpu.VMEM((2, _CH, Hl), jnp.int32),
            pltpu.VMEM((_CH,), jnp.int32),
            pltpu.VMEM((_CH,), jnp.int32),
            pltpu.VMEM((_NBUF, L, D), f32),
            pltpu.VMEM((Vh, D), f32),
            pltpu.VMEM((_CH, D), f32),
            pltpu.VMEM((_CH, D), f32),
            pltpu.VMEM((_CH, D), f32),
            pltpu.VMEM((_CH, D), f32),
        ] + [pltpu.SemaphoreType.DMA] * (_NBUF + 1),
    )
    def k(oid_h, osz_h, lix_h, hix_h, otab_h, stab_h, htab_h,
          id_out, sz_out, lm_out, hm_out,
          lix_v, hix_v, oid_v, osz_v, lbuf, htab_v, idbuf, szbuf, lmv, hmv,
          *allsems):
        sems = list(allsems[:_NBUF])
        sem_e = allsems[_NBUF]
        wid = lax.axis_index("s") * NC + lax.axis_index("c")
        base = wid * RPT

        pltpu.sync_copy(htab_h, htab_v)

        def stage(c):
            scd = c % 2
            row0 = base + c * _CH
            pltpu.sync_copy(lix_h.at[pl.ds(row0, _CH)], lix_v.at[scd])
            pltpu.sync_copy(hix_h.at[pl.ds(row0, _CH)], hix_v.at[scd])
            pltpu.sync_copy(oid_h.at[pl.ds(row0, _CH)], oid_v)
            pltpu.sync_copy(osz_h.at[pl.ds(row0, _CH)], osz_v)
            ca = pltpu.async_copy(otab_h.at[oid_v], idbuf, sem_e)
            cb = pltpu.async_copy(stab_h.at[osz_v], szbuf, sem_e)
            ca.wait()
            cb.wait()
            pltpu.sync_copy(idbuf, id_out.at[pl.ds(row0, _CH)])
            pltpu.sync_copy(szbuf, sz_out.at[pl.ds(row0, _CH)])

        def row_copies(r, b):
            scd = (r // _CH) % 2
            rr = r % _CH
            return tuple(
                pltpu.make_async_copy(
                    otab_h.at[lix_v.at[scd, rr, pl.ds(o, n)]],
                    lbuf.at[b, pl.ds(o, n)], sems[b])
                for o, n in ((0, _SPLIT), (_SPLIT, L2)))

        def issue(r, b):
            for cp in row_copies(r, b):
                cp.start()

        def wait(r, b):
            for cp in row_copies(r, b):
                cp.wait()

        stage(0)
        for b in range(_NBUF - 1):
            issue(b, b)

        z = jnp.zeros((16,), f32)
        lanes = lax.iota(jnp.int32, 16)

        def hist_group(r):
            scd = (r // _CH) % 2
            rr0 = r - (_HG - 1)
            rows = (rr0 % _CH) + lanes
            scds = jnp.full((16,), scd, jnp.int32)

            def hj(j, accs):
                js = jnp.full((16,), j, jnp.int32)
                hvec = plsc.load_gather(hix_v, [scds, rows, js])
                return tuple(
                    accs[c] + plsc.load_gather(
                        htab_v, [hvec, jnp.full((16,), c, jnp.int32)])
                    for c in range(D))

            accs = lax.fori_loop(0, Hl, hj, (z,) * D)
            out_rows = (rr0 % _CH) + lanes
            for c in range(D):
                plsc.store_scatter(hmv,
                                   [out_rows, jnp.full((16,), c, jnp.int32)],
                                   accs[c] * inv_h)

        def outer(i, _):
            for b in range(_NBUF):
                r = i * _NBUF + b
                pre = r + _NBUF - 1
                pb = (b + _NBUF - 1) % _NBUF
                wait(r, b)

                @pl.when(jnp.logical_and(pre % _CH == 0, pre < RPT))
                def _():
                    stage(pre // _CH)

                @pl.when(pre < RPT)
                def _():
                    issue(pre, pb)

                def acc_l(j8, carry):
                    accs = list(carry)
                    for u in range(8):
                        j = j8 * 8 + u
                        accs[u % 4] = accs[u % 4] + lbuf[b, j, pl.ds(0, 16)]
                        accs[4 + u % 4] = (accs[4 + u % 4]
                                           + lbuf[b, j, pl.ds(16, 16)])
                    return tuple(accs)

                accs = lax.fori_loop(0, L // 8, acc_l, (z,) * 8)
                a0 = (accs[0] + accs[1]) + (accs[2] + accs[3])
                a1 = (accs[4] + accs[5]) + (accs[6] + accs[7])
                lmv[r % _CH, pl.ds(0, 16)] = a0 * inv_l
                lmv[r % _CH, pl.ds(16, 16)] = a1 * inv_l

                @pl.when(r % _HG == _HG - 1)
                def _():
                    hist_group(r)

                @pl.when(r % _CH == _CH - 1)
                def _():
                    row0 = base + (r // _CH) * _CH
                    pltpu.sync_copy(lmv, lm_out.at[pl.ds(row0, _CH)])
                    pltpu.sync_copy(hmv, hm_out.at[pl.ds(row0, _CH)])
            return 0

        lax.fori_loop(0, RPT // _NBUF, outer, 0)

    return k(obj_id, obj_size, lines_idx, hist_idx,
             obj_table, size_table, hist_table)


def _lstm_tc_body(id_ref, sz_ref, lm_ref, hm_ref, w1_ref, w2_ref, b_ref,
                  out_ref):
    g = (jnp.dot(id_ref[...], w1_ref[...], preferred_element_type=jnp.float32)
         + jnp.dot(sz_ref[...], w2_ref[...], preferred_element_type=jnp.float32)
         + b_ref[...])
    Hh = g.shape[1] // 3
    i = jax.nn.sigmoid(g[:, :Hh])
    gg = jnp.tanh(g[:, Hh:2 * Hh])
    o = jax.nn.sigmoid(g[:, 2 * Hh:])
    h1 = o * jnp.tanh(i * gg)
    out_ref[:, :Hh] = h1
    D = lm_ref.shape[1]
    out_ref[:, Hh:Hh + D] = lm_ref[...]
    out_ref[:, Hh + D:Hh + 2 * D] = hm_ref[...]


def kernel(obj_id, obj_size, cache_lines, cache_history, obj_id_table,
           obj_size_table, history_table, W_ih, W_hh, b_ih, b_hh):
    B = obj_id.shape[0]
    D = obj_id_table.shape[1]
    Hh = W_hh.shape[1]

    id_emb, sz_emb, lines_mean, hist_mean = _sc_gather_means(
        obj_id, obj_size, cache_lines, cache_history,
        obj_id_table, obj_size_table, history_table)

    Wt = W_ih.T
    Wk = jnp.concatenate([Wt[:, :Hh], Wt[:, 2 * Hh:]], axis=1)
    bk = (b_ih + b_hh)
    bk = jnp.concatenate([bk[:Hh], bk[2 * Hh:]])[None, :]
    w1, w2 = Wk[:D], Wk[D:]

    BM = 2048
    grid = (B // BM,)
    out = pl.pallas_call(
        _lstm_tc_body,
        grid=grid,
        in_specs=[
            pl.BlockSpec((BM, D), lambda i: (i, 0)),
            pl.BlockSpec((BM, D), lambda i: (i, 0)),
            pl.BlockSpec((BM, D), lambda i: (i, 0)),
            pl.BlockSpec((BM, D), lambda i: (i, 0)),
            pl.BlockSpec((D, 3 * Hh), lambda i: (0, 0)),
            pl.BlockSpec((D, 3 * Hh), lambda i: (0, 0)),
            pl.BlockSpec((1, 3 * Hh), lambda i: (0, 0)),
        ],
        out_specs=pl.BlockSpec((BM, Hh + 2 * D), lambda i: (i, 0)),
        out_shape=jax.ShapeDtypeStruct((B, Hh + 2 * D), jnp.float32),
    )(id_emb, sz_emb, lines_mean, hist_mean, w1, w2, bk)
    return out
